# Initial kernel scaffold; baseline (speedup 1.0000x reference)
import jax
import jax.numpy as jnp
from jax.experimental import pallas as pl


def kernel(boxes, classification, rotation, translation):
    raise NotImplementedError("write your pallas kernel here")


# speedup vs baseline: 32.0065x; 32.0065x over previous
import jax
import jax.numpy as jnp
import numpy as np
from jax.experimental import pallas as pl
from jax.experimental.pallas import tpu as pltpu

_N = 5000
_C = 4
_BK = 128
_NPAD = 5120
_NB = _NPAD // _BK
_SCORE_THRESHOLD = 0.01
_NMS_THRESHOLD = 0.5
_MAX_DETECTIONS = 100


def _nms_blocked_kernel(x1r, y1r, x2r, y2r, vr,
                        x1c, y1c, x2c, y2c,
                        kept_out, m_ref, sup_ref):
    f32 = jnp.float32
    lane = jax.lax.broadcasted_iota(jnp.int32, (1, _BK), 1)
    subl = jax.lax.broadcasted_iota(jnp.int32, (_BK, 1), 0)
    pos = jax.lax.broadcasted_iota(jnp.int32, (1, _NPAD), 1)

    sup_ref[...] = jnp.zeros((1, _NPAD), f32)

    def jbody(j, _):
        off = j * _BK
        b_x1r = x1r[0, :, pl.ds(off, _BK)]
        b_y1r = y1r[0, :, pl.ds(off, _BK)]
        b_x2r = x2r[0, :, pl.ds(off, _BK)]
        b_y2r = y2r[0, :, pl.ds(off, _BK)]
        b_val = vr[0, :, pl.ds(off, _BK)]
        b_x1c = x1c[0, pl.ds(off, _BK), :]
        b_y1c = y1c[0, pl.ds(off, _BK), :]
        b_x2c = x2c[0, pl.ds(off, _BK), :]
        b_y2c = y2c[0, pl.ds(off, _BK), :]

        areas_br = (b_x2r - b_x1r) * (b_y2r - b_y1r)
        areas_bc = (b_x2c - b_x1c) * (b_y2c - b_y1c)

        xx1 = jnp.maximum(b_x1c, b_x1r)
        yy1 = jnp.maximum(b_y1c, b_y1r)
        xx2 = jnp.minimum(b_x2c, b_x2r)
        yy2 = jnp.minimum(b_y2c, b_y2r)
        w = jnp.maximum(0.0, xx2 - xx1)
        h = jnp.maximum(0.0, yy2 - yy1)
        inter = w * h
        iou = inter / (areas_bc + areas_br - inter)
        m_ref[...] = (iou > _NMS_THRESHOLD).astype(f32)

        supb0 = sup_ref[:, pl.ds(off, _BK)]

        def ibody(i, st):
            supb, keptb, keptc = st
            is_i = lane == i
            sup_i = jnp.max(jnp.where(is_i, supb, 0.0))
            val_i = jnp.max(jnp.where(is_i, b_val, 0.0))
            take = val_i * (1.0 - sup_i)
            keptb = jnp.where(is_i, take, keptb)
            keptc = jnp.where(subl == i, take, keptc)
            mrow = m_ref[pl.ds(i, 1), :]
            later = (lane > i).astype(f32)
            supb = jnp.maximum(supb, take * mrow * later)
            return supb, keptb, keptc

        supb, keptb, keptc = jax.lax.fori_loop(
            0, _BK, ibody,
            (supb0, jnp.zeros((1, _BK), f32), jnp.zeros((_BK, 1), f32)))

        kept_out[0, :, pl.ds(off, _BK)] = keptb

        x1 = x1r[0]
        y1 = y1r[0]
        x2 = x2r[0]
        y2 = y2r[0]
        areas_all = (x2 - x1) * (y2 - y1)
        cx1 = jnp.maximum(b_x1c, x1)
        cy1 = jnp.maximum(b_y1c, y1)
        cx2 = jnp.minimum(b_x2c, x2)
        cy2 = jnp.minimum(b_y2c, y2)
        cw = jnp.maximum(0.0, cx2 - cx1)
        ch = jnp.maximum(0.0, cy2 - cy1)
        cinter = cw * ch
        ciou = cinter / (areas_bc + areas_all - cinter)
        hits = keptc * (ciou > _NMS_THRESHOLD).astype(f32)
        contrib = jnp.max(hits, axis=0, keepdims=True)
        gate = (pos >= off + _BK).astype(f32)
        sup_ref[...] = jnp.maximum(sup_ref[...], contrib * gate)
        return 0

    jax.lax.fori_loop(0, _NB, jbody, 0)


def _run_nms(sboxes, valid):
    x1 = sboxes[..., 0]
    y1 = sboxes[..., 1]
    x2 = sboxes[..., 2]
    y2 = sboxes[..., 3]
    row_spec = pl.BlockSpec((1, 1, _NPAD), lambda c: (c, 0, 0))
    col_spec = pl.BlockSpec((1, _NPAD, 1), lambda c: (c, 0, 0))
    kept = pl.pallas_call(
        _nms_blocked_kernel,
        grid=(_C,),
        in_specs=[row_spec, row_spec, row_spec, row_spec, row_spec,
                  col_spec, col_spec, col_spec, col_spec],
        out_specs=row_spec,
        out_shape=jax.ShapeDtypeStruct((_C, 1, _NPAD), jnp.float32),
        scratch_shapes=[pltpu.VMEM((_BK, _BK), jnp.float32),
                        pltpu.VMEM((1, _NPAD), jnp.float32)],
    )(x1[:, None], y1[:, None], x2[:, None], y2[:, None], valid[:, None],
      x1[..., None], y1[..., None], x2[..., None], y2[..., None])
    return kept[:, 0]


def kernel(boxes, classification, rotation, translation):
    lab_dtype = jnp.asarray(np.int64(0)).dtype
    b = boxes[0]
    cls = classification[0]
    rot = rotation[0]
    tra = translation[0]

    scT = cls.T
    order = jnp.argsort(-scT, axis=-1, stable=True)
    ssc = jnp.take_along_axis(scT, order, axis=-1)
    sboxes = b[order]
    valid = (ssc > _SCORE_THRESHOLD).astype(jnp.float32)

    pad = _NPAD - _N
    sboxes_p = jnp.pad(sboxes, ((0, 0), (0, pad), (0, 0)))
    valid_p = jnp.pad(valid, ((0, 0), (0, pad)))

    kept_f = _run_nms(sboxes_p, valid_p)
    kept = kept_f[:, :_N] > 0.5

    neg_inf = jnp.asarray(-jnp.inf, ssc.dtype)
    keys = jnp.where(kept, ssc, neg_inf).reshape(-1)
    idxs = order.reshape(-1)
    labs = jnp.repeat(jnp.arange(_C, dtype=lab_dtype), _N)

    gorder = jnp.argsort(-keys, stable=True)
    top = gorder[:_MAX_DETECTIONS]
    n_valid = jnp.sum(kept)
    slot = jnp.arange(_MAX_DETECTIONS)
    ok = slot < n_valid
    sel_idx = jnp.where(ok, idxs[top], 0)
    sel_lab = jnp.where(ok, labs[top], 0)

    out_b = jnp.where(ok[:, None], b[sel_idx], jnp.asarray(-1.0, b.dtype))
    out_s = jnp.where(ok, cls[sel_idx, sel_lab],
                      jnp.asarray(-1.0, cls.dtype))
    out_l = jnp.where(ok, sel_lab, jnp.asarray(-1, lab_dtype))
    out_r = jnp.where(ok[:, None], rot[sel_idx],
                      jnp.asarray(-1.0, rot.dtype))
    out_t = jnp.where(ok[:, None], tra[sel_idx],
                      jnp.asarray(-1.0, tra.dtype))
    return (out_b[None], out_s[None], out_l[None],
            out_r[None], out_t[None])


# speedup vs baseline: 32.5790x; 1.0179x over previous
import jax
import jax.numpy as jnp
import numpy as np
from jax.experimental import pallas as pl
from jax.experimental.pallas import tpu as pltpu

_N = 5000
_C = 4
_BK = 128
_NPAD = 5120
_NB = _NPAD // _BK
_SCORE_THRESHOLD = 0.01
_NMS_THRESHOLD = 0.5
_MAX_DETECTIONS = 100


def _nms_blocked_kernel(x1r, y1r, x2r, y2r, vr,
                        x1c, y1c, x2c, y2c,
                        kept_out, m_ref, sup_ref):
    f32 = jnp.float32
    lane = jax.lax.broadcasted_iota(jnp.int32, (1, _BK), 1)
    subl = jax.lax.broadcasted_iota(jnp.int32, (_BK, 1), 0)
    pos = jax.lax.broadcasted_iota(jnp.int32, (1, _NPAD), 1)

    sup_ref[...] = jnp.zeros((1, _NPAD), f32)

    def jbody(j, _):
        off = j * _BK
        b_x1r = x1r[0, :, pl.ds(off, _BK)]
        b_y1r = y1r[0, :, pl.ds(off, _BK)]
        b_x2r = x2r[0, :, pl.ds(off, _BK)]
        b_y2r = y2r[0, :, pl.ds(off, _BK)]
        b_val = vr[0, :, pl.ds(off, _BK)]
        b_x1c = x1c[0, pl.ds(off, _BK), :]
        b_y1c = y1c[0, pl.ds(off, _BK), :]
        b_x2c = x2c[0, pl.ds(off, _BK), :]
        b_y2c = y2c[0, pl.ds(off, _BK), :]

        areas_br = (b_x2r - b_x1r) * (b_y2r - b_y1r)
        areas_bc = (b_x2c - b_x1c) * (b_y2c - b_y1c)

        xx1 = jnp.maximum(b_x1c, b_x1r)
        yy1 = jnp.maximum(b_y1c, b_y1r)
        xx2 = jnp.minimum(b_x2c, b_x2r)
        yy2 = jnp.minimum(b_y2c, b_y2r)
        w = jnp.maximum(0.0, xx2 - xx1)
        h = jnp.maximum(0.0, yy2 - yy1)
        inter = w * h
        iou = inter / (areas_bc + areas_br - inter)
        tri = (lane > subl).astype(f32)
        m_ref[...] = (iou > _NMS_THRESHOLD).astype(f32) * tri

        supb0 = jnp.maximum(sup_ref[:, pl.ds(off, _BK)], 1.0 - b_val)

        def ibody(i, st):
            supb, keptc = st
            is_i = lane == i
            sup_i = jnp.max(jnp.where(is_i, supb, 0.0))
            take = 1.0 - sup_i
            keptc = jnp.where(subl == i, take, keptc)
            mrow = m_ref[pl.ds(i, 1), :]
            supb = jnp.maximum(supb, take * mrow)
            return supb, keptc

        supb, keptc = jax.lax.fori_loop(
            0, _BK, ibody,
            (supb0, jnp.zeros((_BK, 1), f32)))

        kept_out[0, :, pl.ds(off, _BK)] = b_val * (1.0 - supb)

        x1 = x1r[0]
        y1 = y1r[0]
        x2 = x2r[0]
        y2 = y2r[0]
        areas_all = (x2 - x1) * (y2 - y1)
        cx1 = jnp.maximum(b_x1c, x1)
        cy1 = jnp.maximum(b_y1c, y1)
        cx2 = jnp.minimum(b_x2c, x2)
        cy2 = jnp.minimum(b_y2c, y2)
        cw = jnp.maximum(0.0, cx2 - cx1)
        ch = jnp.maximum(0.0, cy2 - cy1)
        cinter = cw * ch
        ciou = cinter / (areas_bc + areas_all - cinter)
        hits = keptc * (ciou > _NMS_THRESHOLD).astype(f32)
        contrib = jnp.max(hits, axis=0, keepdims=True)
        gate = (pos >= off + _BK).astype(f32)
        sup_ref[...] = jnp.maximum(sup_ref[...], contrib * gate)
        return 0

    jax.lax.fori_loop(0, _NB, jbody, 0)


def _run_nms(sboxes, valid):
    x1 = sboxes[..., 0]
    y1 = sboxes[..., 1]
    x2 = sboxes[..., 2]
    y2 = sboxes[..., 3]
    row_spec = pl.BlockSpec((1, 1, _NPAD), lambda c: (c, 0, 0))
    col_spec = pl.BlockSpec((1, _NPAD, 1), lambda c: (c, 0, 0))
    kept = pl.pallas_call(
        _nms_blocked_kernel,
        grid=(_C,),
        in_specs=[row_spec, row_spec, row_spec, row_spec, row_spec,
                  col_spec, col_spec, col_spec, col_spec],
        out_specs=row_spec,
        out_shape=jax.ShapeDtypeStruct((_C, 1, _NPAD), jnp.float32),
        scratch_shapes=[pltpu.VMEM((_BK, _BK), jnp.float32),
                        pltpu.VMEM((1, _NPAD), jnp.float32)],
    )(x1[:, None], y1[:, None], x2[:, None], y2[:, None], valid[:, None],
      x1[..., None], y1[..., None], x2[..., None], y2[..., None])
    return kept[:, 0]


def kernel(boxes, classification, rotation, translation):
    lab_dtype = jnp.asarray(np.int64(0)).dtype
    b = boxes[0]
    cls = classification[0]
    rot = rotation[0]
    tra = translation[0]

    scT = cls.T
    order = jnp.argsort(-scT, axis=-1, stable=True)
    ssc = jnp.take_along_axis(scT, order, axis=-1)
    sboxes = b[order]
    valid = (ssc > _SCORE_THRESHOLD).astype(jnp.float32)

    pad = _NPAD - _N
    sboxes_p = jnp.pad(sboxes, ((0, 0), (0, pad), (0, 0)))
    valid_p = jnp.pad(valid, ((0, 0), (0, pad)))

    kept_f = _run_nms(sboxes_p, valid_p)
    kept = kept_f[:, :_N] > 0.5

    neg_inf = jnp.asarray(-jnp.inf, ssc.dtype)
    keys = jnp.where(kept, ssc, neg_inf).reshape(-1)
    idxs = order.reshape(-1)
    labs = jnp.repeat(jnp.arange(_C, dtype=lab_dtype), _N)

    gorder = jnp.argsort(-keys, stable=True)
    top = gorder[:_MAX_DETECTIONS]
    n_valid = jnp.sum(kept)
    slot = jnp.arange(_MAX_DETECTIONS)
    ok = slot < n_valid
    sel_idx = jnp.where(ok, idxs[top], 0)
    sel_lab = jnp.where(ok, labs[top], 0)

    out_b = jnp.where(ok[:, None], b[sel_idx], jnp.asarray(-1.0, b.dtype))
    out_s = jnp.where(ok, cls[sel_idx, sel_lab],
                      jnp.asarray(-1.0, cls.dtype))
    out_l = jnp.where(ok, sel_lab, jnp.asarray(-1, lab_dtype))
    out_r = jnp.where(ok[:, None], rot[sel_idx],
                      jnp.asarray(-1.0, rot.dtype))
    out_t = jnp.where(ok[:, None], tra[sel_idx],
                      jnp.asarray(-1.0, tra.dtype))
    return (out_b[None], out_s[None], out_l[None],
            out_r[None], out_t[None])
